# baseline (speedup 1.0000x reference)
import functools

import jax
import jax.numpy as jnp
from jax import lax
from jax.experimental import pallas as pl
from jax.experimental.pallas import tpu as pltpu
from jax.experimental.pallas import tpu_sc as plsc


def _token_sum_sc(ids2d, ktab, BN, L):
    H = ktab.shape[1]
    NW = 32
    npw = BN // NW
    rows_per_chunk = 128
    nodes_per_chunk = rows_per_chunk // L
    chunks = npw // nodes_per_chunk
    id_rows_per_w = npw * L // 128

    mesh = plsc.VectorSubcoreMesh(core_axis_name="c", subcore_axis_name="s")

    @functools.partial(
        pl.kernel, mesh=mesh,
        out_type=jax.ShapeDtypeStruct((BN, H), jnp.float32),
        scratch_types=[
            pltpu.VMEM((id_rows_per_w, 128), jnp.int32),
            pltpu.VMEM((rows_per_chunk, H), jnp.float32),
            pltpu.VMEM((rows_per_chunk, H), jnp.float32),
            pltpu.VMEM((nodes_per_chunk, H), jnp.float32),
            pltpu.SemaphoreType.DMA,
            pltpu.SemaphoreType.DMA,
        ],
    )
    def k(ids_hbm, ktab_hbm, out_hbm, idx_v, rows0, rows1, out_v,
          sem0, sem1):
        cid = lax.axis_index("c")
        sid = lax.axis_index("s")
        wid = cid * 16 + sid
        node_base = wid * npw
        pltpu.sync_copy(ids_hbm.at[pl.ds(wid * id_rows_per_w, id_rows_per_w)],
                        idx_v)

        def fire(c, buf, sem):
            pltpu.async_copy(ktab_hbm.at[idx_v.at[c]], buf, sem)

        def drain(c, buf, sem):
            pltpu.make_async_copy(ktab_hbm.at[idx_v.at[c]], buf, sem).wait()

        def reduce_chunk(c, buf):
            def node_body(nl, carry):
                for dd in range(H // 16):
                    acc = buf[nl * L, pl.ds(dd * 16, 16)]
                    for l in range(1, L):
                        acc = acc + buf[nl * L + l, pl.ds(dd * 16, 16)]
                    out_v[nl, pl.ds(dd * 16, 16)] = acc
                return carry
            lax.fori_loop(0, nodes_per_chunk, node_body, 0)
            pltpu.sync_copy(
                out_v,
                out_hbm.at[pl.ds(node_base + c * nodes_per_chunk,
                                 nodes_per_chunk)])

        fire(0, rows0, sem0)

        def body(g, carry):
            c0 = g * 2
            c1 = g * 2 + 1
            drain(c0, rows0, sem0)
            fire(c1, rows1, sem1)
            reduce_chunk(c0, rows0)
            drain(c1, rows1, sem1)

            @pl.when(g < chunks // 2 - 1)
            def _():
                fire(c1 + 1, rows0, sem0)

            reduce_chunk(c1, rows1)
            return carry

        lax.fori_loop(0, chunks // 2, body, 0)

    return k(ids2d, ktab)


def _dotT(a, b):
    return lax.dot_general(a, b, (((0,), (0,)), ((), ())),
                           preferred_element_type=jnp.float32)


def _block_body(nidx_ref, ntype_ref, toksum_ref, ci_ref, cinc_ref, ttab_ref,
                ptab_ref, posW_ref, posb_ref, wta_ref, wtb_ref, wla_ref,
                wlb_ref, wra_ref, wrb_ref, bias_ref, gamma_ref, beta_ref,
                out_ref):
    f32 = jnp.float32
    i32 = jnp.int32
    ntype = ntype_ref[0]
    nidx = nidx_ref[0]
    tok_sum = toksum_ref[0]
    ci = ci_ref[0]
    ci_nc = cinc_ref[0]
    C, N = ci.shape
    TVp, H = ttab_ref.shape
    Pp, D = ptab_ref.shape

    iota_tv = lax.broadcasted_iota(i32, (TVp, 1), 0)
    t_ohT = (iota_tv == ntype).astype(f32)
    type_emb = _dotT(t_ohT, ttab_ref[...])

    posw = jnp.dot(ptab_ref[...], posW_ref[...], preferred_element_type=f32)
    iota_p = lax.broadcasted_iota(i32, (Pp, 1), 0)
    p_ohT = (iota_p == nidx).astype(f32)
    pos_l = _dotT(p_ohT, posw[:, :H])
    pos_r = _dotT(p_ohT, posw[:, H:])

    pL = type_emb + pos_l + posb_ref[:, :H]
    pR = tok_sum + pos_r + posb_ref[:, H:]

    iota_n = lax.broadcasted_iota(i32, (N, 1), 0)
    PT = jnp.zeros((N, N), f32)
    for c in range(C):
        eq = iota_n == ci[c][None, :]
        PT = PT + jnp.where(eq, float(4096 + c), 0.0)
    AT = jnp.floor(PT * (1.0 / 4096.0))
    KT = PT - 4096.0 * AT

    ns = jnp.sum((ci_nc != 0).astype(f32), axis=1, keepdims=True)
    single = ns == 1.0
    at0 = (ci_nc[:, 0:1] != 0).astype(f32)
    alpha = jnp.where(single, 0.5 * at0, 0.0)
    beta = jnp.where(single, 0.0, 1.0 / jnp.where(single, 1.0, ns - 1.0))

    rowmask = (iota_n != 0).astype(f32)
    lL = pL * rowmask
    lR = pR * rowmask

    Q_l = (jnp.dot(lL, wla_ref[...], preferred_element_type=f32)
           + jnp.dot(lR, wlb_ref[...], preferred_element_type=f32))
    Q_r = (jnp.dot(lL, wra_ref[...], preferred_element_type=f32)
           + jnp.dot(lR, wrb_ref[...], preferred_element_type=f32))
    G = Q_r - Q_l
    children = (_dotT(AT, Q_l) + alpha * _dotT(AT, G) + beta * _dotT(KT, G))

    res = (jnp.dot(pL, wta_ref[...], preferred_element_type=f32)
           + jnp.dot(pR, wtb_ref[...], preferred_element_type=f32)
           + children
           + bias_ref[...])

    mu = jnp.mean(res, axis=1, keepdims=True)
    var = jnp.mean((res - mu) ** 2, axis=1, keepdims=True)
    res = (res - mu) / jnp.sqrt(var + 1e-5) * gamma_ref[...] + beta_ref[...]
    res = jnp.maximum(res, 0.0)
    out_ref[0] = jnp.max(res, axis=0)[None, :]


def kernel(node_index, node_type_index, node_height, node_token_ids,
           children_index, type_table, token_table, pos_table, pos_W, pos_b,
           w_t, w_l, w_r, bias, ln_gamma, ln_beta):
    del node_height
    B, N = node_index.shape
    C = children_index.shape[2]
    L = node_token_ids.shape[2]
    TV, H = type_table.shape
    P = pos_table.shape[0]
    D = pos_W.shape[0]
    f32 = jnp.float32

    def rup(x, mult):
        return ((x + mult - 1) // mult) * mult

    TVp = rup(TV, 8)
    Pp = rup(P, 8)
    ttab = jnp.pad(type_table, ((0, TVp - TV), (0, 0)))
    ptab = jnp.pad(pos_table, ((0, Pp - P), (0, 0)))

    ktab = token_table.astype(f32)
    nidx_all = node_index.astype(jnp.int32).reshape(B, 1, N)
    ntype_all = node_type_index.astype(jnp.int32).reshape(B, 1, N)
    ci_all = children_index.astype(jnp.int32).transpose(0, 2, 1)
    cinc_all = children_index.astype(jnp.int32)

    row = lambda v: v.reshape(1, D).astype(f32)
    wsplit = lambda w: (w.astype(f32)[:H], w.astype(f32)[H:])
    wt_a, wt_b = wsplit(w_t)
    wl_a, wl_b = wsplit(w_l)
    wr_a, wr_b = wsplit(w_r)

    full2 = lambda s1, s2: pl.BlockSpec((s1, s2), lambda b: (0, 0))

    S = 2
    Bs = B // S

    def tc_call(s):
        off3 = lambda s1, s2: pl.BlockSpec((1, s1, s2),
                                           lambda b: (b + s * Bs, 0, 0))
        loc3 = lambda s1, s2: pl.BlockSpec((1, s1, s2), lambda b: (b, 0, 0))
        return pl.pallas_call(
            _block_body,
            grid=(Bs,),
            in_specs=[
                off3(1, N),
                off3(1, N),
                loc3(N, H),
                off3(C, N),
                off3(N, C),
                full2(TVp, H),
                full2(Pp, D),
                full2(D, D),
                full2(1, D),
                full2(H, D),
                full2(H, D),
                full2(H, D),
                full2(H, D),
                full2(H, D),
                full2(H, D),
                full2(1, D),
                full2(1, D),
                full2(1, D),
            ],
            out_specs=pl.BlockSpec((1, 1, D), lambda b: (b, 0, 0)),
            out_shape=jax.ShapeDtypeStruct((Bs, 1, D), f32),
            compiler_params=pltpu.CompilerParams(
                dimension_semantics=("arbitrary",)),
        )

    tok_sums = []
    for s in range(S):
        sl = slice(s * Bs, (s + 1) * Bs)
        ids2d = node_token_ids[sl].astype(jnp.int32).reshape(
            Bs * N * L // 128, 128)
        tok_sums.append(_token_sum_sc(ids2d, ktab, Bs * N, L).reshape(Bs, N, H))

    outs = []
    for s in range(S):
        outs.append(tc_call(s)(
            nidx_all, ntype_all, tok_sums[s], ci_all, cinc_all, ttab, ptab,
            pos_W.astype(f32), row(pos_b), wt_a, wt_b, wl_a, wl_b, wr_a, wr_b,
            row(bias), row(ln_gamma), row(ln_beta)))
    return jnp.concatenate(outs, axis=0).reshape(B, D)


# speedup vs baseline: 1.0280x; 1.0280x over previous
import functools

import jax
import jax.numpy as jnp
from jax import lax
from jax.experimental import pallas as pl
from jax.experimental.pallas import tpu as pltpu
from jax.experimental.pallas import tpu_sc as plsc


def _token_sum_sc(ids2d, ktab, BN, L):
    H = ktab.shape[1]
    NW = 32
    npw = BN // NW
    rows_per_chunk = 128
    nodes_per_chunk = rows_per_chunk // L
    chunks = npw // nodes_per_chunk
    id_rows_per_w = npw * L // 128

    mesh = plsc.VectorSubcoreMesh(core_axis_name="c", subcore_axis_name="s")

    @functools.partial(
        pl.kernel, mesh=mesh,
        out_type=jax.ShapeDtypeStruct((BN, H), jnp.float32),
        scratch_types=[
            pltpu.VMEM((id_rows_per_w, 128), jnp.int32),
            pltpu.VMEM((rows_per_chunk, H), jnp.float32),
            pltpu.VMEM((rows_per_chunk, H), jnp.float32),
            pltpu.VMEM((nodes_per_chunk, H), jnp.float32),
            pltpu.SemaphoreType.DMA,
            pltpu.SemaphoreType.DMA,
        ],
    )
    def k(ids_hbm, ktab_hbm, out_hbm, idx_v, rows0, rows1, out_v,
          sem0, sem1):
        cid = lax.axis_index("c")
        sid = lax.axis_index("s")
        wid = cid * 16 + sid
        node_base = wid * npw
        pltpu.sync_copy(ids_hbm.at[pl.ds(wid * id_rows_per_w, id_rows_per_w)],
                        idx_v)

        def fire(c, buf, sem):
            pltpu.async_copy(ktab_hbm.at[idx_v.at[c]], buf, sem)

        def drain(c, buf, sem):
            pltpu.make_async_copy(ktab_hbm.at[idx_v.at[c]], buf, sem).wait()

        def reduce_chunk(c, buf):
            def node_body(nl, carry):
                for dd in range(H // 16):
                    acc = buf[nl * L, pl.ds(dd * 16, 16)]
                    for l in range(1, L):
                        acc = acc + buf[nl * L + l, pl.ds(dd * 16, 16)]
                    out_v[nl, pl.ds(dd * 16, 16)] = acc
                return carry
            lax.fori_loop(0, nodes_per_chunk, node_body, 0)
            pltpu.sync_copy(
                out_v,
                out_hbm.at[pl.ds(node_base + c * nodes_per_chunk,
                                 nodes_per_chunk)])

        fire(0, rows0, sem0)

        def body(g, carry):
            c0 = g * 2
            c1 = g * 2 + 1
            drain(c0, rows0, sem0)
            fire(c1, rows1, sem1)
            reduce_chunk(c0, rows0)
            drain(c1, rows1, sem1)

            @pl.when(g < chunks // 2 - 1)
            def _():
                fire(c1 + 1, rows0, sem0)

            reduce_chunk(c1, rows1)
            return carry

        lax.fori_loop(0, chunks // 2, body, 0)

    return k(ids2d, ktab)


def _dotT(a, b):
    return lax.dot_general(a, b, (((0,), (0,)), ((), ())),
                           preferred_element_type=jnp.float32)


def _block_body(nidx_ref, ntype_ref, toksum_ref, ci_ref, cinc_ref, ttab_ref,
                ptab_ref, posW_ref, posb_ref, wta_ref, wtb_ref, wla_ref,
                wlb_ref, wra_ref, wrb_ref, bias_ref, gamma_ref, beta_ref,
                out_ref):
    f32 = jnp.float32
    i32 = jnp.int32
    ntype = ntype_ref[0]
    nidx = nidx_ref[0]
    tok_sum = toksum_ref[0]
    ci = ci_ref[0]
    ci_nc = cinc_ref[0]
    C, N = ci.shape
    TVp, H = ttab_ref.shape
    Pp, D = ptab_ref.shape

    iota_tv = lax.broadcasted_iota(i32, (TVp, 1), 0)
    t_ohT = (iota_tv == ntype).astype(f32)
    type_emb = _dotT(t_ohT, ttab_ref[...])

    posw = jnp.dot(ptab_ref[...], posW_ref[...], preferred_element_type=f32)
    iota_p = lax.broadcasted_iota(i32, (Pp, 1), 0)
    p_ohT = (iota_p == nidx).astype(f32)
    pos_l = _dotT(p_ohT, posw[:, :H])
    pos_r = _dotT(p_ohT, posw[:, H:])

    pL = type_emb + pos_l + posb_ref[:, :H]
    pR = tok_sum + pos_r + posb_ref[:, H:]

    iota_n = lax.broadcasted_iota(i32, (N, 1), 0)
    PT = jnp.zeros((N, N), f32)
    for c in range(C):
        eq = iota_n == ci[c][None, :]
        PT = PT + jnp.where(eq, float(4096 + c), 0.0)
    AT = jnp.floor(PT * (1.0 / 4096.0))
    KT = PT - 4096.0 * AT

    ns = jnp.sum((ci_nc != 0).astype(f32), axis=1, keepdims=True)
    single = ns == 1.0
    at0 = (ci_nc[:, 0:1] != 0).astype(f32)
    alpha = jnp.where(single, 0.5 * at0, 0.0)
    beta = jnp.where(single, 0.0, 1.0 / jnp.where(single, 1.0, ns - 1.0))

    rowmask = (iota_n != 0).astype(f32)
    lL = pL * rowmask
    lR = pR * rowmask

    Q_l = (jnp.dot(lL, wla_ref[...], preferred_element_type=f32)
           + jnp.dot(lR, wlb_ref[...], preferred_element_type=f32))
    Q_r = (jnp.dot(lL, wra_ref[...], preferred_element_type=f32)
           + jnp.dot(lR, wrb_ref[...], preferred_element_type=f32))
    G = Q_r - Q_l
    children = (_dotT(AT, Q_l) + alpha * _dotT(AT, G) + beta * _dotT(KT, G))

    res = (jnp.dot(pL, wta_ref[...], preferred_element_type=f32)
           + jnp.dot(pR, wtb_ref[...], preferred_element_type=f32)
           + children
           + bias_ref[...])

    mu = jnp.mean(res, axis=1, keepdims=True)
    var = jnp.mean((res - mu) ** 2, axis=1, keepdims=True)
    res = (res - mu) / jnp.sqrt(var + 1e-5) * gamma_ref[...] + beta_ref[...]
    res = jnp.maximum(res, 0.0)
    out_ref[0] = jnp.max(res, axis=0)[None, :]


def kernel(node_index, node_type_index, node_height, node_token_ids,
           children_index, type_table, token_table, pos_table, pos_W, pos_b,
           w_t, w_l, w_r, bias, ln_gamma, ln_beta):
    del node_height
    B, N = node_index.shape
    C = children_index.shape[2]
    L = node_token_ids.shape[2]
    TV, H = type_table.shape
    P = pos_table.shape[0]
    D = pos_W.shape[0]
    f32 = jnp.float32

    def rup(x, mult):
        return ((x + mult - 1) // mult) * mult

    TVp = rup(TV, 8)
    Pp = rup(P, 8)
    ttab = jnp.pad(type_table, ((0, TVp - TV), (0, 0)))
    ptab = jnp.pad(pos_table, ((0, Pp - P), (0, 0)))

    ktab = token_table.astype(f32)
    nidx_all = node_index.astype(jnp.int32).reshape(B, 1, N)
    ntype_all = node_type_index.astype(jnp.int32).reshape(B, 1, N)
    ci_all = children_index.astype(jnp.int32).transpose(0, 2, 1)
    cinc_all = children_index.astype(jnp.int32)

    row = lambda v: v.reshape(1, D).astype(f32)
    wsplit = lambda w: (w.astype(f32)[:H], w.astype(f32)[H:])
    wt_a, wt_b = wsplit(w_t)
    wl_a, wl_b = wsplit(w_l)
    wr_a, wr_b = wsplit(w_r)

    full2 = lambda s1, s2: pl.BlockSpec((s1, s2), lambda b: (0, 0))

    S = 4
    Bs = B // S

    def tc_call(s):
        off3 = lambda s1, s2: pl.BlockSpec((1, s1, s2),
                                           lambda b: (b + s * Bs, 0, 0))
        loc3 = lambda s1, s2: pl.BlockSpec((1, s1, s2), lambda b: (b, 0, 0))
        return pl.pallas_call(
            _block_body,
            grid=(Bs,),
            in_specs=[
                off3(1, N),
                off3(1, N),
                loc3(N, H),
                off3(C, N),
                off3(N, C),
                full2(TVp, H),
                full2(Pp, D),
                full2(D, D),
                full2(1, D),
                full2(H, D),
                full2(H, D),
                full2(H, D),
                full2(H, D),
                full2(H, D),
                full2(H, D),
                full2(1, D),
                full2(1, D),
                full2(1, D),
            ],
            out_specs=pl.BlockSpec((1, 1, D), lambda b: (b, 0, 0)),
            out_shape=jax.ShapeDtypeStruct((Bs, 1, D), f32),
            compiler_params=pltpu.CompilerParams(
                dimension_semantics=("arbitrary",)),
        )

    tok_sums = []
    for s in range(S):
        sl = slice(s * Bs, (s + 1) * Bs)
        ids2d = node_token_ids[sl].astype(jnp.int32).reshape(
            Bs * N * L // 128, 128)
        tok_sums.append(_token_sum_sc(ids2d, ktab, Bs * N, L).reshape(Bs, N, H))

    outs = []
    for s in range(S):
        outs.append(tc_call(s)(
            nidx_all, ntype_all, tok_sums[s], ci_all, cinc_all, ttab, ptab,
            pos_W.astype(f32), row(pos_b), wt_a, wt_b, wl_a, wl_b, wr_a, wr_b,
            row(bias), row(ln_gamma), row(ln_beta)))
    return jnp.concatenate(outs, axis=0).reshape(B, D)
